# baseline (speedup 1.0000x reference)
import functools

import jax
import jax.numpy as jnp
from jax import lax
from jax.experimental import pallas as pl
from jax.experimental.pallas import tpu as pltpu
from jax.experimental.pallas import tpu_sc as plsc

D = 64
SCALE = 8.0
B_TOTAL = 4096 * 200
NC = 2
NS = 16
NW = NC * NS
B_PER_W = B_TOTAL // NW
CHUNK = 128
NBUF = 4
N_CHUNKS = B_PER_W // CHUNK

assert B_PER_W * NW == B_TOTAL
assert N_CHUNKS * CHUNK == B_PER_W
assert (N_CHUNKS - 2 * NBUF) % NBUF == 0


def _scale_chunk(rin, rout, b):
    def row(r, carry):
        for j in range(D // 16):
            s = pl.ds(j * 16, 16)
            rout[b, r, s] = rin[b, r, s] * SCALE
        return carry
    lax.fori_loop(0, CHUNK, row, 0)


def _make_emb():
    mesh = plsc.VectorSubcoreMesh(core_axis_name="c", subcore_axis_name="s")

    @functools.partial(
        pl.kernel,
        mesh=mesh,
        out_type=jax.ShapeDtypeStruct((B_TOTAL, 2 * D), jnp.float32),
        compiler_params=pltpu.CompilerParams(use_tc_tiling_on_sc=False),
        scratch_types=(
            [pltpu.VMEM((B_PER_W,), jnp.int32),
             pltpu.VMEM((NBUF, CHUNK, D), jnp.float32),
             pltpu.VMEM((NBUF, CHUNK, D), jnp.float32)]
            + [pltpu.SemaphoreType.DMA] * (1 + 2 * NBUF)
        ),
    )
    def emb(x_hbm, table_hbm, out_hbm, idx_v, rin_v, rout_v, *sems):
        isem = sems[0]
        gsem = sems[1:1 + NBUF]
        ssem = sems[1 + NBUF:]
        wid = lax.axis_index("s") * NC + lax.axis_index("c")
        base = wid * B_PER_W

        pltpu.async_copy(x_hbm.at[pl.ds(base, B_PER_W)], idx_v, isem)
        pltpu.make_async_copy(
            x_hbm.at[pl.ds(base, B_PER_W)], idx_v, isem).wait()

        def issue_gather(g, b):
            pltpu.async_copy(
                table_hbm.at[idx_v.at[pl.ds(g * CHUNK, CHUNK)]],
                rin_v.at[b], gsem[b])

        def wait_gather(b):
            pltpu.make_async_copy(
                table_hbm.at[idx_v.at[pl.ds(0, CHUNK)]], rin_v.at[b],
                gsem[b]).wait()

        def issue_scatter(g, b):
            off = base + g * CHUNK
            pltpu.async_copy(
                rout_v.at[b],
                out_hbm.at[pl.ds(off, CHUNK), pl.ds(0, D)], ssem[b])

        def wait_scatter(g, b):
            off = base + g * CHUNK
            pltpu.make_async_copy(
                rout_v.at[b],
                out_hbm.at[pl.ds(off, CHUNK), pl.ds(0, D)], ssem[b]).wait()

        for b in range(NBUF):
            issue_gather(b, b)

        for b in range(NBUF):
            wait_gather(b)
            _scale_chunk(rin_v, rout_v, b)
            issue_scatter(b, b)
            issue_gather(b + NBUF, b)

        def outer(i, carry):
            g0 = NBUF + i * NBUF
            for b in range(NBUF):
                g = g0 + b
                wait_gather(b)
                wait_scatter(g, b)
                _scale_chunk(rin_v, rout_v, b)
                issue_scatter(g, b)
                issue_gather(g + NBUF, b)
            return carry
        lax.fori_loop(0, (N_CHUNKS - 2 * NBUF) // NBUF, outer, 0)

        for b in range(NBUF):
            g = N_CHUNKS - NBUF + b
            wait_gather(b)
            wait_scatter(g, b)
            _scale_chunk(rin_v, rout_v, b)
            issue_scatter(g, b)

        for b in range(NBUF):
            wait_scatter(N_CHUNKS - NBUF + b, b)

    return emb


_emb = _make_emb()

NG = 1_000_000 // 8
RCH = 20
NCH_TOT = NG // RCH
RMAX = (NCH_TOT + NW - 1) // NW
assert RMAX % 4 == 0


def _make_linearize():
    mesh = plsc.VectorSubcoreMesh(core_axis_name="c", subcore_axis_name="s")

    @functools.partial(
        pl.kernel,
        mesh=mesh,
        out_type=jax.ShapeDtypeStruct((500000, 128), jnp.float32),
        compiler_params=pltpu.CompilerParams(use_tc_tiling_on_sc=True,
                                             needs_layout_passes=False),
        scratch_types=(
            [pltpu.VMEM((4, RCH, 8, D), jnp.float32),
             pltpu.VMEM((4, RCH * 4, 128), jnp.float32)]
            + [pltpu.SemaphoreType.DMA] * 8
        ),
    )
    def linearize(table_hbm, out_hbm, a_v, b_v, *sems):
        rsem = sems[:4]
        wsem = sems[4:]
        wid = lax.axis_index("s") * NC + lax.axis_index("c")
        lo = wid * NCH_TOT // NW
        hi = (wid + 1) * NCH_TOT // NW

        def rd(g, sl):
            pltpu.async_copy(
                table_hbm.at[pl.ds((lo + g) * RCH, RCH)], a_v.at[sl],
                rsem[sl])

        def rd_wait(sl):
            pltpu.make_async_copy(
                table_hbm.at[pl.ds(0, RCH)], a_v.at[sl], rsem[sl]).wait()

        def wr(g, sl):
            pltpu.async_copy(
                b_v.at[sl],
                out_hbm.at[pl.ds((lo + g) * RCH * 4, RCH * 4)], wsem[sl])

        def wr_wait(sl):
            pltpu.make_async_copy(
                b_v.at[sl], out_hbm.at[pl.ds(0, RCH * 4)], wsem[sl]).wait()

        def repack(sl):
            def body(gq, carry):
                for h in range(4):
                    q = gq * 4 + h
                    for half in range(2):
                        s = 2 * h + half
                        for j0 in range(0, D, 16):
                            v = a_v[sl, gq, s, pl.ds(j0, 16)]
                            b_v[sl, q, pl.ds(half * D + j0, 16)] = v
                return carry
            lax.fori_loop(0, RCH, body, 0)

        for sl in range(4):
            rd(sl, sl)

        def outer(i, carry):
            for k in range(4):
                g = 4 * i + k
                c = lo + g

                @pl.when(c < hi)
                def _a():
                    rd_wait(k)

                @pl.when(jnp.logical_and(g >= 4, c - 4 < hi))
                def _b():
                    wr_wait(k)

                @pl.when(c < hi)
                def _c():
                    repack(k)
                    wr(g, k)

                @pl.when(c + 4 < hi)
                def _d():
                    rd(g + 4, k)
            return carry
        lax.fori_loop(0, RMAX // 4, outer, 0)

        for k in range(4):
            @pl.when(lo + RMAX - 4 + k < hi)
            def _e():
                wr_wait(k)

    return linearize


_linearize = _make_linearize()


def kernel(x, table):
    tab_lin = _linearize(table.reshape(NG, 8, D))
    out = _emb(x.reshape(B_TOTAL).astype(jnp.int32),
               tab_lin.reshape(1_000_000, D))
    return out[:, :D].reshape(4096, 200, D)


# speedup vs baseline: 1.0621x; 1.0621x over previous
import functools

import jax
import jax.numpy as jnp
from jax import lax
from jax.experimental import pallas as pl
from jax.experimental.pallas import tpu as pltpu
from jax.experimental.pallas import tpu_sc as plsc

D = 64
SCALE = 8.0
B_TOTAL = 4096 * 200
NC = 2
NS = 16
NW = NC * NS
B_PER_W = B_TOTAL // NW
CHUNK = 128
NBUF = 4
N_CHUNKS = B_PER_W // CHUNK

assert B_PER_W * NW == B_TOTAL
assert N_CHUNKS * CHUNK == B_PER_W
assert (N_CHUNKS - 2 * NBUF) % NBUF == 0


def _scale_chunk(rin, rout, b):
    def row(r, carry):
        for j in range(D // 16):
            s = pl.ds(j * 16, 16)
            rout[b, r, s] = rin[b, r, s] * SCALE
        return carry
    lax.fori_loop(0, CHUNK, row, 0)


def _make_emb():
    mesh = plsc.VectorSubcoreMesh(core_axis_name="c", subcore_axis_name="s")

    @functools.partial(
        pl.kernel,
        mesh=mesh,
        out_type=jax.ShapeDtypeStruct((B_TOTAL, 2 * D), jnp.float32),
        compiler_params=pltpu.CompilerParams(use_tc_tiling_on_sc=False),
        scratch_types=(
            [pltpu.VMEM((B_PER_W,), jnp.int32),
             pltpu.VMEM((NBUF, CHUNK, D), jnp.float32),
             pltpu.VMEM((NBUF, CHUNK, D), jnp.float32)]
            + [pltpu.SemaphoreType.DMA] * (1 + 2 * NBUF)
        ),
    )
    def emb(x_hbm, table_hbm, out_hbm, idx_v, rin_v, rout_v, *sems):
        isem = sems[0]
        gsem = sems[1:1 + NBUF]
        ssem = sems[1 + NBUF:]
        wid = lax.axis_index("s") * NC + lax.axis_index("c")
        base = wid * B_PER_W

        pltpu.async_copy(x_hbm.at[pl.ds(base, B_PER_W)], idx_v, isem)
        pltpu.make_async_copy(
            x_hbm.at[pl.ds(base, B_PER_W)], idx_v, isem).wait()

        def issue_gather(g, b):
            pltpu.async_copy(
                table_hbm.at[idx_v.at[pl.ds(g * CHUNK, CHUNK)]],
                rin_v.at[b], gsem[b])

        def wait_gather(b):
            pltpu.make_async_copy(
                table_hbm.at[idx_v.at[pl.ds(0, CHUNK)]], rin_v.at[b],
                gsem[b]).wait()

        def issue_scatter(g, b):
            off = base + g * CHUNK
            pltpu.async_copy(
                rout_v.at[b],
                out_hbm.at[pl.ds(off, CHUNK), pl.ds(0, D)], ssem[b])

        def wait_scatter(g, b):
            off = base + g * CHUNK
            pltpu.make_async_copy(
                rout_v.at[b],
                out_hbm.at[pl.ds(off, CHUNK), pl.ds(0, D)], ssem[b]).wait()

        for b in range(NBUF):
            issue_gather(b, b)

        for b in range(NBUF):
            wait_gather(b)
            _scale_chunk(rin_v, rout_v, b)
            issue_scatter(b, b)
            issue_gather(b + NBUF, b)

        def outer(i, carry):
            g0 = NBUF + i * NBUF
            for b in range(NBUF):
                g = g0 + b
                wait_gather(b)
                wait_scatter(g, b)
                _scale_chunk(rin_v, rout_v, b)
                issue_scatter(g, b)
                issue_gather(g + NBUF, b)
            return carry
        lax.fori_loop(0, (N_CHUNKS - 2 * NBUF) // NBUF, outer, 0)

        for b in range(NBUF):
            g = N_CHUNKS - NBUF + b
            wait_gather(b)
            wait_scatter(g, b)
            _scale_chunk(rin_v, rout_v, b)
            issue_scatter(g, b)

        for b in range(NBUF):
            wait_scatter(N_CHUNKS - NBUF + b, b)

    return emb


_emb = _make_emb()


def kernel(x, table):
    out = _emb(x.reshape(B_TOTAL).astype(jnp.int32), table)
    return out[:, :D].reshape(4096, 200, D)
